# baseline (speedup 1.0000x reference)
import functools

import jax
import jax.numpy as jnp
from jax import lax
from jax.experimental import pallas as pl
from jax.experimental.pallas import tpu as pltpu
from jax.experimental.pallas import tpu_sc as plsc

NC, NS = 2, 16
NW = NC * NS
BCHUNK = 128
RING = 4
GLAG = 2


def _emb_body(idx_hbm, table_hbm, out_hbm, idx_v, g_ring, t_ring, gsem, wsem,
              *, hist, b_per_w):
    wid = lax.axis_index("s") * NC + lax.axis_index("c")
    b0 = wid * b_per_w
    n_c = b_per_w // BCHUNK
    n_steps = hist * n_c

    pltpu.sync_copy(idx_hbm.at[:, pl.ds(b0, b_per_w)], idx_v)

    rows = [lax.iota(jnp.int32, 16) + 16 * i for i in range(BCHUNK // 16)]

    def _fire_gather(step, slot):
        h = step // n_c
        c = lax.rem(step, n_c)
        pltpu.async_copy(
            table_hbm.at[idx_v.at[h, pl.ds(c * BCHUNK, BCHUNK)]],
            g_ring[slot], gsem.at[slot],
        )

    def _fire_write(step, slot):
        h = step // n_c
        c = lax.rem(step, n_c)
        pltpu.async_copy(
            t_ring[slot],
            out_hbm.at[h, :, pl.ds(b0 + c * BCHUNK, BCHUNK)],
            wsem.at[slot],
        )

    def _wait_gather(slot):
        pltpu.make_async_copy(
            table_hbm.at[pl.ds(0, BCHUNK)], g_ring[slot], gsem.at[slot]
        ).wait()

    def _wait_write(slot):
        pltpu.make_async_copy(
            t_ring[slot], out_hbm.at[0, :, pl.ds(0, BCHUNK)], wsem.at[slot]
        ).wait()

    def _transpose(gbuf, tbuf):
        @pl.loop(0, 64, unroll=8)
        def _(d):
            col = jnp.full((16,), d, jnp.int32)
            for i in range(BCHUNK // 16):
                v = plsc.load_gather(gbuf, [rows[i], col])
                tbuf[d, pl.ds(16 * i, 16)] = v

    for s in range(GLAG):
        _fire_gather(s, s)

    @pl.loop(0, n_steps, step=RING)
    def _(s0):
        for b in range(RING):
            slot = b

            @pl.when(s0 + b >= RING)
            def _():
                _wait_write(slot)

            nxt = s0 + b + GLAG

            @pl.when(nxt < n_steps)
            def _():
                _fire_gather(nxt, (b + GLAG) % RING)

            _wait_gather(slot)
            _transpose(g_ring[slot], t_ring[slot])
            _fire_write(s0 + b, slot)

    for s in range(RING):
        _wait_write(s)


def kernel(inputs, weight):
    bsz, hist = inputs.shape
    vocab, dim = weight.shape
    assert bsz % (NW * BCHUNK) == 0 and dim == 64
    b_per_w = bsz // NW

    idx_t = jnp.transpose(inputs).astype(jnp.int32)

    run = pl.kernel(
        functools.partial(_emb_body, hist=hist, b_per_w=b_per_w),
        out_type=jax.ShapeDtypeStruct((hist, dim, bsz), jnp.float32),
        mesh=plsc.VectorSubcoreMesh(
            core_axis_name="c", subcore_axis_name="s",
            num_cores=NC, num_subcores=NS,
        ),
        scratch_types=[
            pltpu.VMEM((hist, b_per_w), jnp.int32),
            [pltpu.VMEM((BCHUNK, dim), jnp.float32) for _ in range(RING)],
            [pltpu.VMEM((dim, BCHUNK), jnp.float32) for _ in range(RING)],
            pltpu.SemaphoreType.DMA((RING,)),
            pltpu.SemaphoreType.DMA((RING,)),
        ],
        compiler_params=pltpu.CompilerParams(
            use_tc_tiling_on_sc=False, needs_layout_passes=False
        ),
    )
    raw = run(idx_t, weight)
    return jnp.transpose(raw, (2, 0, 1))


# speedup vs baseline: 1.1359x; 1.1359x over previous
import functools

import jax
import jax.numpy as jnp
from jax import lax
from jax.experimental import pallas as pl
from jax.experimental.pallas import tpu as pltpu
from jax.experimental.pallas import tpu_sc as plsc

NC, NS = 2, 16
NW = NC * NS
BCHUNK = 128
RING = 4
GLAG = 2


def _emb_body(idx_hbm, table_hbm, out_hbm, idx_v, g_ring, t_ring, gsem, wsem,
              *, hist, b_per_w):
    wid = lax.axis_index("s") * NC + lax.axis_index("c")
    b0 = wid * b_per_w
    n_c = b_per_w // BCHUNK
    n_steps = hist * n_c

    pltpu.sync_copy(idx_hbm.at[:, pl.ds(b0, b_per_w)], idx_v)

    rows = [lax.iota(jnp.int32, 16) + 16 * i for i in range(BCHUNK // 16)]

    def _fire_gather(step, slot):
        h = step // n_c
        c = lax.rem(step, n_c)
        pltpu.async_copy(
            table_hbm.at[idx_v.at[h, pl.ds(c * BCHUNK, BCHUNK)]],
            g_ring[slot], gsem.at[slot],
        )

    def _fire_write(step, slot):
        h = step // n_c
        c = lax.rem(step, n_c)
        pltpu.async_copy(
            t_ring[slot],
            out_hbm.at[h, :, pl.ds(b0 + c * BCHUNK, BCHUNK)],
            wsem.at[slot],
        )

    def _wait_gather(slot):
        pltpu.make_async_copy(
            table_hbm.at[pl.ds(0, BCHUNK)], g_ring[slot], gsem.at[slot]
        ).wait()

    def _wait_write(slot):
        pltpu.make_async_copy(
            t_ring[slot], out_hbm.at[0, :, pl.ds(0, BCHUNK)], wsem.at[slot]
        ).wait()

    def _transpose(gbuf, tbuf):
        @pl.loop(0, 64, unroll=4)
        def _(d):
            col = jnp.full((16,), d, jnp.int32)
            vs = [plsc.load_gather(gbuf, [rows[i], col])
                  for i in range(BCHUNK // 16)]
            for i, v in enumerate(vs):
                tbuf[d, pl.ds(16 * i, 16)] = v

    for s in range(GLAG):
        _fire_gather(s, s)

    @pl.loop(0, n_steps, step=RING)
    def _(s0):
        for b in range(RING):
            slot = b

            @pl.when(s0 + b >= RING)
            def _():
                _wait_write(slot)

            nxt = s0 + b + GLAG

            @pl.when(nxt < n_steps)
            def _():
                _fire_gather(nxt, (b + GLAG) % RING)

            _wait_gather(slot)
            _transpose(g_ring[slot], t_ring[slot])
            _fire_write(s0 + b, slot)

    for s in range(RING):
        _wait_write(s)


def kernel(inputs, weight):
    bsz, hist = inputs.shape
    vocab, dim = weight.shape
    assert bsz % (NW * BCHUNK) == 0 and dim == 64
    b_per_w = bsz // NW

    idx_t = jnp.transpose(inputs).astype(jnp.int32)

    run = pl.kernel(
        functools.partial(_emb_body, hist=hist, b_per_w=b_per_w),
        out_type=jax.ShapeDtypeStruct((hist, dim, bsz), jnp.float32),
        mesh=plsc.VectorSubcoreMesh(
            core_axis_name="c", subcore_axis_name="s",
            num_cores=NC, num_subcores=NS,
        ),
        scratch_types=[
            pltpu.VMEM((hist, b_per_w), jnp.int32),
            [pltpu.VMEM((BCHUNK, dim), jnp.float32) for _ in range(RING)],
            [pltpu.VMEM((dim, BCHUNK), jnp.float32) for _ in range(RING)],
            pltpu.SemaphoreType.DMA((RING,)),
            pltpu.SemaphoreType.DMA((RING,)),
        ],
        compiler_params=pltpu.CompilerParams(
            use_tc_tiling_on_sc=False, needs_layout_passes=False
        ),
    )
    raw = run(idx_t, weight)
    return jnp.transpose(raw, (2, 0, 1))


# speedup vs baseline: 1.2878x; 1.1337x over previous
import functools

import jax
import jax.numpy as jnp
from jax import lax
from jax.experimental import pallas as pl
from jax.experimental.pallas import tpu as pltpu
from jax.experimental.pallas import tpu_sc as plsc

NC, NS = 2, 16
NW = NC * NS
BCHUNK = 128
RING = 4
GLAG = 2


def _emb_body(idx_hbm, table_hbm, out_hbm, idx_v, g_ring, t_ring, gsem, wsem,
              *, hist, b_per_w):
    wid = lax.axis_index("s") * NC + lax.axis_index("c")
    b0 = wid * b_per_w
    n_c = b_per_w // BCHUNK
    n_steps = hist * n_c

    pltpu.sync_copy(idx_hbm.at[:, pl.ds(b0, b_per_w)], idx_v)

    rows = [lax.iota(jnp.int32, 16) + 16 * i for i in range(BCHUNK // 16)]

    def _fire_gather(step, slot):
        h = step // n_c
        c = lax.rem(step, n_c)
        pltpu.async_copy(
            table_hbm.at[idx_v.at[h, pl.ds(c * BCHUNK, BCHUNK)]],
            g_ring[slot], gsem.at[slot],
        )

    def _fire_write(step, slot):
        h = step // n_c
        c = lax.rem(step, n_c)
        blk = (b0 // BCHUNK) + c
        pltpu.async_copy(
            t_ring[slot],
            out_hbm.at[h, :, blk],
            wsem.at[slot],
        )

    def _wait_gather(slot):
        pltpu.make_async_copy(
            table_hbm.at[pl.ds(0, BCHUNK)], g_ring[slot], gsem.at[slot]
        ).wait()

    def _wait_write(slot):
        pltpu.make_async_copy(
            t_ring[slot], out_hbm.at[0, :, 0], wsem.at[slot]
        ).wait()

    def _transpose(gbuf, tbuf):
        @pl.loop(0, 64, unroll=4)
        def _(d):
            col = jnp.full((16,), d, jnp.int32)
            vs = [plsc.load_gather(gbuf, [rows[i], col])
                  for i in range(BCHUNK // 16)]
            g_hi = lax.div(d, 8)
            g_lo = lax.rem(d, 8)
            for i, v in enumerate(vs):
                tbuf[g_hi, g_lo, pl.ds(16 * i, 16)] = v

    for s in range(GLAG):
        _fire_gather(s, s)

    @pl.loop(0, n_steps, step=RING)
    def _(s0):
        for b in range(RING):
            slot = b

            @pl.when(s0 + b >= RING)
            def _():
                _wait_write(slot)

            nxt = s0 + b + GLAG

            @pl.when(nxt < n_steps)
            def _():
                _fire_gather(nxt, (b + GLAG) % RING)

            _wait_gather(slot)
            _transpose(g_ring[slot], t_ring[slot])
            _fire_write(s0 + b, slot)

    for s in range(RING):
        _wait_write(s)


def kernel(inputs, weight):
    bsz, hist = inputs.shape
    vocab, dim = weight.shape
    assert bsz % (NW * BCHUNK) == 0 and dim == 64
    b_per_w = bsz // NW

    idx_t = jnp.transpose(inputs).astype(jnp.int32)

    run = pl.kernel(
        functools.partial(_emb_body, hist=hist, b_per_w=b_per_w),
        out_type=jax.ShapeDtypeStruct(
            (hist, dim // 8, bsz // 128, 8, 128), jnp.float32
        ),
        mesh=plsc.VectorSubcoreMesh(
            core_axis_name="c", subcore_axis_name="s",
            num_cores=NC, num_subcores=NS,
        ),
        scratch_types=[
            pltpu.VMEM((hist, b_per_w), jnp.int32),
            [pltpu.VMEM((BCHUNK, dim), jnp.float32) for _ in range(RING)],
            [pltpu.VMEM((dim // 8, 8, BCHUNK), jnp.float32) for _ in range(RING)],
            pltpu.SemaphoreType.DMA((RING,)),
            pltpu.SemaphoreType.DMA((RING,)),
        ],
        compiler_params=pltpu.CompilerParams(
            use_tc_tiling_on_sc=False, needs_layout_passes=False
        ),
    )
    raw = run(idx_t, weight)
    out = jnp.transpose(raw, (2, 4, 0, 1, 3)).reshape(bsz, hist, dim)
    return out


# speedup vs baseline: 2.5362x; 1.9693x over previous
import functools

import jax
import jax.numpy as jnp
from jax import lax
from jax.experimental import pallas as pl
from jax.experimental.pallas import tpu as pltpu
from jax.experimental.pallas import tpu_sc as plsc

NC, NS = 2, 16
NW = NC * NS
BCHUNK = 128
RING = 4
GLAG = 2


def _emb_body(idx_hbm, table_hbm, out_hbm, idx_v, g_ring, t_ring, gsem, wsem,
              *, hist, b_per_w):
    wid = lax.axis_index("s") * NC + lax.axis_index("c")
    b0 = wid * b_per_w
    n_c = b_per_w // BCHUNK
    n_steps = hist * n_c

    pltpu.sync_copy(idx_hbm.at[:, pl.ds(b0, b_per_w)], idx_v)

    lane = lax.iota(jnp.int32, 16)
    rows = [lane + 16 * i for i in range(BCHUNK // 16)]

    def _fire_gather(step, slot):
        h = step // n_c
        c = lax.rem(step, n_c)
        pltpu.async_copy(
            table_hbm.at[idx_v.at[h, pl.ds(c * BCHUNK, BCHUNK)]],
            g_ring[slot], gsem.at[slot],
        )

    def _fire_write(step, slot):
        h = step // n_c
        c = lax.rem(step, n_c)
        blk = (b0 // BCHUNK) + c
        pltpu.async_copy(
            t_ring[slot],
            out_hbm.at[h, :, blk],
            wsem.at[slot],
        )

    def _wait_gather(slot):
        pltpu.make_async_copy(
            table_hbm.at[pl.ds(0, BCHUNK)], g_ring[slot], gsem.at[slot]
        ).wait()

    def _wait_write(slot):
        pltpu.make_async_copy(
            t_ring[slot], out_hbm.at[0, :, 0], wsem.at[slot]
        ).wait()

    def _transpose(gbuf, tbuf):
        for dblk in range(4):
            @pl.loop(0, 16, unroll=4)
            def _(k):
                d_vec = ((lane + k) & 15) + 16 * dblk
                g_hi = lax.shift_right_logical(d_vec, 3)
                g_lo = d_vec & 7
                vs = [plsc.load_gather(gbuf, [rows[i], d_vec])
                      for i in range(BCHUNK // 16)]
                for i, v in enumerate(vs):
                    plsc.store_scatter(tbuf, [g_hi, g_lo, rows[i]], v)

    for s in range(GLAG):
        _fire_gather(s, s)

    @pl.loop(0, n_steps, step=RING)
    def _(s0):
        for b in range(RING):
            slot = b

            @pl.when(s0 + b >= RING)
            def _():
                _wait_write(slot)

            nxt = s0 + b + GLAG

            @pl.when(nxt < n_steps)
            def _():
                _fire_gather(nxt, (b + GLAG) % RING)

            _wait_gather(slot)
            _transpose(g_ring[slot], t_ring[slot])
            _fire_write(s0 + b, slot)

    for s in range(RING):
        _wait_write(s)


def kernel(inputs, weight):
    bsz, hist = inputs.shape
    vocab, dim = weight.shape
    assert bsz % (NW * BCHUNK) == 0 and dim == 64
    b_per_w = bsz // NW

    idx_t = jnp.transpose(inputs).astype(jnp.int32)

    run = pl.kernel(
        functools.partial(_emb_body, hist=hist, b_per_w=b_per_w),
        out_type=jax.ShapeDtypeStruct(
            (hist, dim // 8, bsz // 128, 8, 128), jnp.float32
        ),
        mesh=plsc.VectorSubcoreMesh(
            core_axis_name="c", subcore_axis_name="s",
            num_cores=NC, num_subcores=NS,
        ),
        scratch_types=[
            pltpu.VMEM((hist, b_per_w), jnp.int32),
            [pltpu.VMEM((BCHUNK, dim), jnp.float32) for _ in range(RING)],
            [pltpu.VMEM((dim // 8, 8, BCHUNK), jnp.float32) for _ in range(RING)],
            pltpu.SemaphoreType.DMA((RING,)),
            pltpu.SemaphoreType.DMA((RING,)),
        ],
        compiler_params=pltpu.CompilerParams(
            use_tc_tiling_on_sc=False, needs_layout_passes=False
        ),
    )
    raw = run(idx_t, weight)
    out = jnp.transpose(raw, (2, 4, 0, 1, 3)).reshape(bsz, hist, dim)
    return out


# speedup vs baseline: 3.2432x; 1.2788x over previous
import functools

import jax
import jax.numpy as jnp
from jax import lax
from jax.experimental import pallas as pl
from jax.experimental.pallas import tpu as pltpu
from jax.experimental.pallas import tpu_sc as plsc

NC, NS = 2, 16
NW = NC * NS
BCHUNK = 128
RING = 4
GLAG = 2
TRING = 3
VCHUNK = 128

_MESH = dict(core_axis_name="c", subcore_axis_name="s",
             num_cores=NC, num_subcores=NS)
_PARAMS = pltpu.CompilerParams(
    use_tc_tiling_on_sc=False, needs_layout_passes=False
)
_PARAMS_TILED = pltpu.CompilerParams(
    use_tc_tiling_on_sc=True, needs_layout_passes=False
)


def _wid():
    return lax.axis_index("s") * NC + lax.axis_index("c")


def _table_body(wt_hbm, wtail_hbm, out_hbm, s_ring, d_ring, s_tail, gsem, wsem,
                *, vocab, dim):
    wid = _wid()
    n_full = vocab // VCHUNK
    tail = vocab - n_full * VCHUNK
    base = n_full // NW
    extra = n_full - base * NW
    start = wid * base + lax.min(wid, extra)
    cnt = base + (wid < extra).astype(jnp.int32)
    steps = base + 2
    assert steps % TRING == 0

    lane = lax.iota(jnp.int32, 16)
    l64 = lane * dim
    vgrp = [lane + 16 * i for i in range(VCHUNK // 16)]

    def _fire_stage(s, r):
        pltpu.async_copy(
            wt_hbm.at[:, pl.ds((start + s) * VCHUNK, VCHUNK)],
            s_ring[r], gsem.at[r],
        )

    def _wait_stage(r):
        pltpu.make_async_copy(
            wt_hbm.at[:, pl.ds(0, VCHUNK)], s_ring[r], gsem.at[r]
        ).wait()

    def _fire_write(s, r):
        pltpu.async_copy(
            d_ring[r],
            out_hbm.at[pl.ds((start + s) * VCHUNK * dim, VCHUNK * dim)],
            wsem.at[r],
        )

    def _wait_write(r):
        pltpu.make_async_copy(
            d_ring[r], out_hbm.at[pl.ds(0, VCHUNK * dim)], wsem.at[r]
        ).wait()

    def _transpose(sbuf, dbuf, nvp):
        @pl.loop(0, 16, unroll=4)
        def _(k):
            doff = (lane + k) & 15
            for dd in range(dim // 16):
                d_vec = doff + 16 * dd
                for vp in range(nvp):
                    val = plsc.load_gather(sbuf, [d_vec, vgrp[vp]])
                    flat = l64 + d_vec + (16 * dim) * vp
                    plsc.store_scatter(dbuf, [flat], val)

    for s in range(GLAG):
        @pl.when(s < cnt)
        def _():
            _fire_stage(s, s)

    @pl.loop(0, steps, step=TRING)
    def _(s0):
        for b in range(TRING):
            s = s0 + b
            r = b

            @pl.when(jnp.logical_and(s >= TRING, s - TRING < cnt))
            def _():
                _wait_write(r)

            nxt = s + GLAG

            @pl.when(nxt < cnt)
            def _():
                _fire_stage(nxt, (b + GLAG) % TRING)

            @pl.when(s < cnt)
            def _():
                _wait_stage(r)
                _transpose(s_ring[r], d_ring[r], VCHUNK // 16)
                _fire_write(s, r)

    for e in range(TRING):
        w = steps - TRING + e

        @pl.when(w < cnt)
        def _():
            _wait_write(w % TRING)

    if tail:
        @pl.when(wid == NW - 1)
        def _():
            pltpu.sync_copy(wtail_hbm, s_tail)
            _transpose(s_tail, d_ring[0], tail // 16)
            pltpu.sync_copy(
                d_ring[0].at[pl.ds(0, tail * dim)],
                out_hbm.at[pl.ds(n_full * VCHUNK * dim, tail * dim)],
            )


def _emb_body(idx_hbm, table_hbm, out_hbm, idx_v, g_ring, t_ring, gsem, wsem,
              *, hist, b_per_w):
    wid = _wid()
    b0 = wid * b_per_w
    n_c = b_per_w // BCHUNK
    n_steps = hist * n_c

    pltpu.sync_copy(idx_hbm.at[:, pl.ds(b0, b_per_w)], idx_v)

    lane = lax.iota(jnp.int32, 16)
    rows = [lane + 16 * i for i in range(BCHUNK // 16)]

    def _fire_gather(step, slot):
        h = step // n_c
        c = lax.rem(step, n_c)
        pltpu.async_copy(
            table_hbm.at[idx_v.at[h, pl.ds(c * BCHUNK, BCHUNK)]],
            g_ring[slot], gsem.at[slot],
        )

    def _fire_write(step, slot):
        h = step // n_c
        c = lax.rem(step, n_c)
        blk = (b0 // BCHUNK) + c
        pltpu.async_copy(t_ring[slot], out_hbm.at[h, :, blk], wsem.at[slot])

    def _wait_gather(slot):
        pltpu.make_async_copy(
            table_hbm.at[pl.ds(0, BCHUNK)], g_ring[slot], gsem.at[slot]
        ).wait()

    def _wait_write(slot):
        pltpu.make_async_copy(
            t_ring[slot], out_hbm.at[0, :, 0], wsem.at[slot]
        ).wait()

    def _transpose(gbuf, tbuf):
        @pl.loop(0, 16, unroll=4)
        def _(k):
            doff = (lane + k) & 15
            for dblk in range(4):
                d_vec = doff + 16 * dblk
                g_hi = lax.shift_right_logical(d_vec, 3)
                g_lo = d_vec & 7
                vs = [plsc.load_gather(gbuf, [rows[i], d_vec])
                      for i in range(BCHUNK // 16)]
                for i, v in enumerate(vs):
                    plsc.store_scatter(tbuf, [g_hi, g_lo, rows[i]], v)

    for s in range(GLAG):
        _fire_gather(s, s)

    @pl.loop(0, n_steps, step=RING)
    def _(s0):
        for b in range(RING):
            slot = b

            @pl.when(s0 + b >= RING)
            def _():
                _wait_write(slot)

            nxt = s0 + b + GLAG

            @pl.when(nxt < n_steps)
            def _():
                _fire_gather(nxt, (b + GLAG) % RING)

            _wait_gather(slot)
            _transpose(g_ring[slot], t_ring[slot])
            _fire_write(s0 + b, slot)

    for s in range(RING):
        _wait_write(s)


def kernel(inputs, weight):
    bsz, hist = inputs.shape
    vocab, dim = weight.shape
    assert bsz % (NW * BCHUNK) == 0 and dim == 64 and vocab % 64 == 0
    b_per_w = bsz // NW

    idx_t = jnp.transpose(inputs).astype(jnp.int32)
    w_t = jnp.transpose(weight)
    tail = vocab % VCHUNK
    w_tail = jnp.transpose(weight[vocab - tail:, :]) if tail else w_t[:, :64]

    to_rows = pl.kernel(
        functools.partial(_table_body, vocab=vocab, dim=dim),
        out_type=jax.ShapeDtypeStruct((vocab * dim,), jnp.float32),
        mesh=plsc.VectorSubcoreMesh(**_MESH),
        scratch_types=[
            [pltpu.VMEM((dim, VCHUNK), jnp.float32) for _ in range(TRING)],
            [pltpu.VMEM((VCHUNK * dim,), jnp.float32) for _ in range(TRING)],
            pltpu.VMEM((dim, max(tail, 16)), jnp.float32),
            pltpu.SemaphoreType.DMA((TRING,)),
            pltpu.SemaphoreType.DMA((TRING,)),
        ],
        compiler_params=_PARAMS_TILED,
    )
    table = to_rows(w_t, w_tail).reshape(vocab, dim)

    run = pl.kernel(
        functools.partial(_emb_body, hist=hist, b_per_w=b_per_w),
        out_type=jax.ShapeDtypeStruct(
            (hist, dim // 8, bsz // 128, 8, 128), jnp.float32
        ),
        mesh=plsc.VectorSubcoreMesh(**_MESH),
        scratch_types=[
            pltpu.VMEM((hist, b_per_w), jnp.int32),
            [pltpu.VMEM((BCHUNK, dim), jnp.float32) for _ in range(RING)],
            [pltpu.VMEM((dim // 8, 8, BCHUNK), jnp.float32) for _ in range(RING)],
            pltpu.SemaphoreType.DMA((RING,)),
            pltpu.SemaphoreType.DMA((RING,)),
        ],
        compiler_params=_PARAMS,
    )
    raw = run(idx_t, table)
    return jnp.transpose(raw, (2, 4, 0, 1, 3)).reshape(bsz, hist, dim)
